# baseline (speedup 1.0000x reference)
import jax
import jax.numpy as jnp
from jax import lax
from jax.experimental import pallas as pl
from jax.experimental.pallas import tpu as pltpu
from jax.experimental.pallas import tpu_sc as plsc

_BATCH = 4096
_HIST = 200
_MAIN_DIM = 16
_SUB_DIM = 48
_OUT_DIM = _MAIN_DIM + _SUB_DIM
_N = _BATCH * _HIST
_NW = 32
_BPW = _BATCH // _NW
_PER_W = _N // _NW
_KA = 104
_KB = 96
_CHUNKS = 2 * _BPW
_NBUF = 4
_GROUPS = _CHUNKS // _NBUF
_KS = (_KA, _KB, _KA, _KB)
_HS = (0, _KA, 0, _KA)
_MT_ROWS = 125


def _blocks(k):
  starts = list(range(0, k - 15, 16))
  if starts[-1] + 16 < k:
    starts.append(k - 16)
  return starts


def _body(mid_hbm, sid_hbm, mt_hbm, st_hbm, out_hbm,
          mtv, sidxs, obufs, orows, mids, isems, gsems, wsems):
  wid = lax.axis_index("s") * 2 + lax.axis_index("c")
  wbase = wid * _PER_W
  b0 = wid * _BPW

  def off(i, b):
    return wbase + (i // 2) * _HIST + _HS[b]

  def idx_start(i, b):
    pltpu.async_copy(sid_hbm.at[pl.ds(off(i, b), _KS[b])],
                     sidxs[b], isems[b])
    pltpu.async_copy(mid_hbm.at[pl.ds(off(i, b), _KS[b])],
                     mids[b], isems[b])

  def idx_wait(b):
    pltpu.make_async_copy(sid_hbm.at[pl.ds(0, _KS[b])],
                          sidxs[b], isems[b]).wait()
    pltpu.make_async_copy(mid_hbm.at[pl.ds(0, _KS[b])],
                          mids[b], isems[b]).wait()

  def gather_start(b):
    k = _KS[b]
    pltpu.async_copy(st_hbm.at[sidxs[b].at[pl.ds(0, k)]],
                     obufs[b], gsems[b])

  def gather_wait(b):
    k = _KS[b]
    pltpu.make_async_copy(st_hbm.at[sidxs[b].at[pl.ds(0, k)]],
                          obufs[b], gsems[b]).wait()

  lanes = lax.iota(jnp.int32, 16)

  def assemble(b):
    k = _KS[b]
    ob128 = obufs[b]
    ob64 = orows[b]

    def rows4(r4, c):
      r = r4 * 4
      for d in range(4):
        ob64[r + d, 16:32] = ob128[r + d, 16:32]
        ob64[r + d, 32:48] = ob128[r + d, 32:48]
        ob64[r + d, 48:64] = ob128[r + d, 48:64]
      return c

    lax.fori_loop(0, k // 4, rows4, 0)

    for r0 in _blocks(k):
     mv = mids[b][pl.ds(r0, 16)]
     rows16 = mv // 8
     colbase = (mv % 8) * 16
     rowidx = r0 + lanes
     for c in range(_MAIN_DIM):
       v = plsc.load_gather(mtv, [rows16, colbase + c])
       plsc.store_scatter(ob64, [rowidx, jnp.full((16,), c, jnp.int32)], v)

  def write_start(i, b):
    k, h = _KS[b], _HS[b]
    bb = b0 + i // 2
    pltpu.async_copy(orows[b], out_hbm.at[bb, pl.ds(h, k), :], wsems[b])

  def write_wait(b):
    k, h = _KS[b], _HS[b]
    pltpu.make_async_copy(orows[b], out_hbm.at[0, pl.ds(h, k), :],
                          wsems[b]).wait()

  pltpu.sync_copy(mt_hbm, mtv)

  for b in range(_NBUF):
    idx_start(b, b)
    if b >= 1:
      idx_wait(b - 1)
      gather_start(b - 1)
    if b >= 2:
      gather_wait(b - 2)
      assemble(b - 2)
    if b >= 3:
      write_start(b - 3, b - 3)

  def group(g, carry):
    for b in range(_NBUF):
      i = g * _NBUF + b
      write_wait(b)
      idx_start(i, b)
      b1 = (b - 1) % _NBUF
      idx_wait(b1)
      gather_start(b1)
      b2 = (b - 2) % _NBUF
      gather_wait(b2)
      assemble(b2)
      b3 = (b - 3) % _NBUF
      write_start(i - 3, b3)
    return carry

  lax.fori_loop(1, _GROUPS, group, 0)

  n = _CHUNKS
  idx_wait((n - 1) % _NBUF)
  gather_start((n - 1) % _NBUF)
  for i in (n - 2, n - 1):
    b = i % _NBUF
    gather_wait(b)
    assemble(b)
  for i in (n - 3, n - 2, n - 1):
    write_start(i, i % _NBUF)
  for j in range(_NBUF):
    write_wait((n - _NBUF + j) % _NBUF)


@jax.jit
def _encode(mid_flat, sid_flat, mt_packed, st_pad):
  mesh = plsc.VectorSubcoreMesh(core_axis_name="c", subcore_axis_name="s")

  def body(mid_hbm, sid_hbm, mt_hbm, st_hbm, out_hbm,
           mtv, si0, si1, si2, si3,
           ob0, ob1, ob2, ob3, or0, or1, or2, or3,
           md0, md1, md2, md3,
           i0, i1, i2, i3, g0, g1, g2, g3, w0, w1, w2, w3):
    _body(mid_hbm, sid_hbm, mt_hbm, st_hbm, out_hbm,
          mtv, (si0, si1, si2, si3),
          (ob0, ob1, ob2, ob3), (or0, or1, or2, or3),
          (md0, md1, md2, md3),
          (i0, i1, i2, i3), (g0, g1, g2, g3), (w0, w1, w2, w3))

  f = pl.kernel(
      body,
      out_type=jax.ShapeDtypeStruct((_BATCH, _HIST, _OUT_DIM), jnp.float32),
      mesh=mesh,
      scratch_types=[
          pltpu.VMEM((_MT_ROWS, 128), jnp.float32),
      ] + [pltpu.VMEM((k,), jnp.int32) for k in _KS]
        + [pltpu.VMEM((k, 128), jnp.float32) for k in _KS]
        + [pltpu.VMEM((k, _OUT_DIM), jnp.float32) for k in _KS]
        + [pltpu.VMEM((k,), jnp.int32) for k in _KS]
        + [pltpu.SemaphoreType.DMA] * (3 * _NBUF),
      compiler_params=pltpu.CompilerParams(use_tc_tiling_on_sc=True, needs_layout_passes=False),
  )
  return f(mid_flat, sid_flat, mt_packed, st_pad)


def kernel(main_category_id, sub_category_id, main_table, sub_table):
  mid = main_category_id.reshape(_N).astype(jnp.int32)
  sid = sub_category_id.reshape(_N).astype(jnp.int32)
  mt_packed = main_table.reshape(_MT_ROWS, 128)
  st_pad = jnp.zeros((sub_table.shape[0], 128), jnp.float32)
  st_pad = st_pad.at[:, _MAIN_DIM:_OUT_DIM].set(sub_table)
  return _encode(mid, sid, mt_packed, st_pad)


# speedup vs baseline: 1.2255x; 1.2255x over previous
import jax
import jax.numpy as jnp
from jax import lax
from jax.experimental import pallas as pl
from jax.experimental.pallas import tpu as pltpu
from jax.experimental.pallas import tpu_sc as plsc

_BATCH = 4096
_HIST = 200
_MAIN_DIM = 16
_SUB_DIM = 48
_OUT_DIM = _MAIN_DIM + _SUB_DIM
_N = _BATCH * _HIST
_NW = 32
_BPW = _BATCH // _NW
_PER_W = _N // _NW
_KA = 104
_KB = 96
_CHUNKS = 2 * _BPW
_NBUF = 4
_GROUPS = _CHUNKS // _NBUF
_KS = (_KA, _KB, _KA, _KB)
_HS = (0, _KA, 0, _KA)


def _body(mid_hbm, sid_hbm, mt_hbm, st_hbm, out_hbm,
          midx_v, sidx_v, mrows, srows, gsems, wsems):
  wid = lax.axis_index("s") * 2 + lax.axis_index("c")
  wbase = wid * _PER_W
  b0 = wid * _BPW

  def gather_start(i, b):
    k, h = _KS[b], _HS[b]
    off = (i // 2) * _HIST + h
    cm = pltpu.async_copy(mt_hbm.at[midx_v.at[pl.ds(off, k)]],
                          mrows[b], gsems[b])
    cs = pltpu.async_copy(st_hbm.at[sidx_v.at[pl.ds(off, k)]],
                          srows[b], gsems[b])
    return cm, cs

  def gather_wait(b):
    k = _KS[b]
    pltpu.make_async_copy(mt_hbm.at[midx_v.at[pl.ds(0, k)]],
                          mrows[b], gsems[b]).wait()
    pltpu.make_async_copy(st_hbm.at[sidx_v.at[pl.ds(0, k)]],
                          srows[b], gsems[b]).wait()

  def write_start(i, b):
    k, h = _KS[b], _HS[b]
    bb = b0 + i // 2
    pltpu.async_copy(mrows[b],
                     out_hbm.at[bb, pl.ds(h, k), 0:_MAIN_DIM], wsems[b])
    pltpu.async_copy(srows[b],
                     out_hbm.at[bb, pl.ds(h, k), _MAIN_DIM:_OUT_DIM],
                     wsems[b])

  def write_wait(b):
    k, h = _KS[b], _HS[b]
    pltpu.make_async_copy(mrows[b],
                          out_hbm.at[0, pl.ds(h, k), 0:_MAIN_DIM],
                          wsems[b]).wait()
    pltpu.make_async_copy(srows[b],
                          out_hbm.at[0, pl.ds(h, k), _MAIN_DIM:_OUT_DIM],
                          wsems[b]).wait()

  pltpu.sync_copy(mid_hbm.at[pl.ds(wbase, _PER_W)], midx_v)
  pltpu.sync_copy(sid_hbm.at[pl.ds(wbase, _PER_W)], sidx_v)

  for b in range(_NBUF):
    gather_start(b, b)
    if b >= 2:
      gather_wait(b - 2)
      write_start(b - 2, b - 2)

  def group(g, carry):
    for b in range(_NBUF):
      i = g * _NBUF + b
      write_wait(b)
      gather_start(i, b)
      bw = (b - 2) % _NBUF
      gather_wait(bw)
      write_start(i - 2, bw)
    return carry

  lax.fori_loop(1, _GROUPS, group, 0)

  for j in range(2):
    i = _CHUNKS - 2 + j
    b = i % _NBUF
    gather_wait(b)
    write_start(i, b)
  for j in range(_NBUF):
    write_wait((_CHUNKS - _NBUF + j) % _NBUF)


@jax.jit
def _encode(mid_flat, sid_flat, main_table, sub_table):
  mesh = plsc.VectorSubcoreMesh(core_axis_name="c", subcore_axis_name="s")

  def body(mid_hbm, sid_hbm, mt_hbm, st_hbm, out_hbm,
           midx_v, sidx_v,
           mr0, mr1, mr2, mr3, sr0, sr1, sr2, sr3,
           g0, g1, g2, g3, w0, w1, w2, w3):
    _body(mid_hbm, sid_hbm, mt_hbm, st_hbm, out_hbm,
          midx_v, sidx_v,
          (mr0, mr1, mr2, mr3), (sr0, sr1, sr2, sr3),
          (g0, g1, g2, g3), (w0, w1, w2, w3))

  f = pl.kernel(
      body,
      out_type=jax.ShapeDtypeStruct((_BATCH, _HIST, _OUT_DIM), jnp.float32),
      mesh=mesh,
      scratch_types=[
          pltpu.VMEM((_PER_W,), jnp.int32),
          pltpu.VMEM((_PER_W,), jnp.int32),
      ] + [pltpu.VMEM((k, _MAIN_DIM), jnp.float32) for k in _KS]
        + [pltpu.VMEM((k, _SUB_DIM), jnp.float32) for k in _KS]
        + [pltpu.SemaphoreType.DMA] * (2 * _NBUF),
      compiler_params=pltpu.CompilerParams(use_tc_tiling_on_sc=False),
  )
  return f(mid_flat, sid_flat, main_table, sub_table)


def kernel(main_category_id, sub_category_id, main_table, sub_table):
  mid = main_category_id.reshape(_N).astype(jnp.int32)
  sid = sub_category_id.reshape(_N).astype(jnp.int32)
  return _encode(mid, sid, main_table, sub_table)
